# Initial kernel scaffold; baseline (speedup 1.0000x reference)
import jax
import jax.numpy as jnp
from jax.experimental import pallas as pl


def kernel(x, W_qkv, W_proj, b_proj):
    raise NotImplementedError("write your pallas kernel here")


# speedup vs baseline: 1.2031x; 1.2031x over previous
import jax
import jax.numpy as jnp
from jax.experimental import pallas as pl

DIM_ = 1024
NHEADS_ = 16
HDIM_ = 64
TBLK_ = 512


def _fused_body(xT_ref, wqkv_ref, wproj_ref, b_ref, out_ref):
    H, D = NHEADS_, HDIM_
    xT = xT_ref[...]
    T = xT.shape[1]
    qkvT = jnp.dot(wqkv_ref[...], xT,
                   preferred_element_type=jnp.float32)
    scale = float(D) ** -0.5
    qT = qkvT[0:DIM_, :] * scale
    kT = qkvT[DIM_:2 * DIM_, :]
    vT = qkvT[2 * DIM_:3 * DIM_, :]
    q3 = qT.reshape(H, D, T)
    k3 = kT.reshape(H, D, T)
    v3 = vT.reshape(H, D, T)
    outs = []
    for h in range(H):
        s_h = jnp.sum(q3[h][None, :, :] * k3, axis=1)
        m = jnp.max(s_h, axis=0, keepdims=True)
        e = jnp.exp(s_h - m)
        r = 1.0 / jnp.sum(e, axis=0, keepdims=True)
        p = e * r
        o_h = jnp.sum(p[:, None, :] * v3, axis=0)
        outs.append(o_h)
    outT = jnp.concatenate(outs, axis=0)
    yT = jnp.dot(wproj_ref[...], outT,
                 preferred_element_type=jnp.float32)
    out_ref[...] = yT + b_ref[...]


def kernel(x, W_qkv, W_proj, b_proj):
    N, C = x.shape
    xT = x.T
    b2 = b_proj.reshape(C, 1)
    grid = (N // TBLK_,)
    yT = pl.pallas_call(
        _fused_body,
        grid=grid,
        in_specs=[
            pl.BlockSpec((C, TBLK_), lambda i: (0, i)),
            pl.BlockSpec((3 * C, C), lambda i: (0, 0)),
            pl.BlockSpec((C, C), lambda i: (0, 0)),
            pl.BlockSpec((C, 1), lambda i: (0, 0)),
        ],
        out_specs=pl.BlockSpec((C, TBLK_), lambda i: (0, i)),
        out_shape=jax.ShapeDtypeStruct((C, N), jnp.float32),
    )(xT, W_qkv, W_proj, b2)
    return yT.T
